# baseline (speedup 1.0000x reference)
import functools

import jax
import jax.numpy as jnp
from jax import lax
from jax.experimental import pallas as pl
from jax.experimental.pallas import tpu as pltpu
from jax.experimental.pallas import tpu_sc as plsc

_N = 32768
_B = 16
_DN = 256
_DM = 256
_NC = 2
_NS = 16
_NW = _NC * _NS
_ROWS = _N // _NW
_CH = 128
_NCHUNK = _ROWS // _CH
_G = _CH // 16


def _sc_body(x_hbm, mr_hbm, wb_hbm, idx_hbm, out_hbm,
             xb0, xb1, w_v, mr_v, ms_v, idx_v, out_v,
             sem0, sem1, sem_s):
    wid = lax.axis_index("s") * _NC + lax.axis_index("c")
    row0 = wid * _ROWS
    lanes = lax.iota(jnp.int32, 16)

    cp_w = pltpu.async_copy(wb_hbm, w_v, sem_s)
    cp_mr = pltpu.async_copy(mr_hbm, mr_v, sem_s)
    cp_idx = pltpu.async_copy(idx_hbm.at[pl.ds(row0, _ROWS)], idx_v, sem_s)
    bufs = [xb0, xb1]
    sems = [sem0, sem1]
    cps = [pltpu.async_copy(x_hbm.at[pl.ds(row0, _CH)], xb0, sem0), None]
    cp_w.wait()
    cp_mr.wait()
    cp_idx.wait()

    def dot_block(ref, rows, woff, k, acc):
        wv = w_v[pl.ds(woff + k * 16, 16)]
        for j in range(16):
            col = plsc.load_gather(
                ref, [rows, jnp.full((16,), k * 16 + j, jnp.int32)])
            acc = acc + col * wv[j]
        return acc

    ms = lax.fori_loop(0, _DM // 16,
                       functools.partial(dot_block, mr_v, lanes, _DN),
                       jnp.zeros(16, jnp.float32))
    ms_v[...] = ms + w_v[pl.ds(_DN + _DM, 16)]

    def compute_chunk(ch, buf):
        def group(g, _):
            rows = g * 16 + lanes
            acc = lax.fori_loop(0, _DN // 16,
                                functools.partial(dot_block, buf, rows, 0),
                                jnp.zeros(16, jnp.float32))
            base = ch * _CH + g * 16
            seg = idx_v[pl.ds(base, 16)]
            logit = acc + plsc.load_gather(ms_v, [seg])
            out_v[pl.ds(base, 16)] = 1.0 / (1.0 + jnp.exp(-logit))
            return 0

        lax.fori_loop(0, _G, group, 0)

    for ch in range(_NCHUNK):
        if ch + 1 < _NCHUNK:
            nxt = (ch + 1) % 2
            cps[nxt] = pltpu.async_copy(
                x_hbm.at[pl.ds(row0 + (ch + 1) * _CH, _CH)], bufs[nxt],
                sems[nxt])
        cps[ch % 2].wait()
        compute_chunk(ch, bufs[ch % 2])

    pltpu.sync_copy(out_v, out_hbm.at[pl.ds(row0, _ROWS)])


@jax.jit
def kernel(node_hiddens, next_motif_mreprs, W, b, batch_indices):
    wb = jnp.concatenate(
        [W[:, 0], jnp.full((16,), b[0], jnp.float32)])
    mesh = plsc.VectorSubcoreMesh(core_axis_name="c", subcore_axis_name="s")
    run = pl.kernel(
        _sc_body,
        out_type=jax.ShapeDtypeStruct((_N,), jnp.float32),
        mesh=mesh,
        scratch_types=[
            pltpu.VMEM((_CH, _DN), jnp.float32),
            pltpu.VMEM((_CH, _DN), jnp.float32),
            pltpu.VMEM((_DN + _DM + 16,), jnp.float32),
            pltpu.VMEM((_B, _DM), jnp.float32),
            pltpu.VMEM((_B,), jnp.float32),
            pltpu.VMEM((_ROWS,), jnp.int32),
            pltpu.VMEM((_ROWS,), jnp.float32),
            pltpu.SemaphoreType.DMA,
            pltpu.SemaphoreType.DMA,
            pltpu.SemaphoreType.DMA,
        ],
        compiler_params=pltpu.CompilerParams(
            use_tc_tiling_on_sc=False, needs_layout_passes=False),
    )
    return run(node_hiddens, next_motif_mreprs, wb, batch_indices)
